# baseline (speedup 1.0000x reference)
import functools

import jax
import jax.numpy as jnp
from jax import lax
from jax.experimental import pallas as pl
from jax.experimental.pallas import tpu as pltpu
from jax.experimental.pallas import tpu_sc as plsc

BLK = 256
G_POOL = 64
F32 = jnp.float32


def _dot(a, b):
    return jnp.dot(a, b, preferred_element_type=F32)


@functools.lru_cache(maxsize=None)
def _make_agg(NP, steps, HH):
    mesh = plsc.VectorSubcoreMesh(core_axis_name="c", subcore_axis_name="s")
    zr = NP // 16

    @functools.partial(
        pl.kernel,
        out_type=jax.ShapeDtypeStruct((2, NP, HH), F32),
        mesh=mesh,
        scratch_types=[
            pltpu.VMEM((128,), jnp.int32),
            pltpu.VMEM((128,), jnp.int32),
            pltpu.VMEM((128, HH), F32),
            pltpu.VMEM_SHARED((NP, HH), F32),
            pltpu.SemaphoreType.DMA,
        ],
    )
    def agg(hf, srcs, dsts, zeros, out, is0, id0, rows0, agg_sh, gs0):
        c = lax.axis_index("c")
        s = lax.axis_index("s")
        pltpu.sync_copy(zeros, agg_sh.at[pl.ds(s * zr, zr)])

        plsc.subcore_barrier()

        def body(g, carry):
            row = s * steps + g
            pltpu.sync_copy(srcs.at[c, row], is0)
            pltpu.sync_copy(dsts.at[row], id0)
            pltpu.async_copy(hf.at[is0], rows0, gs0).wait()
            pltpu.sync_copy(rows0, agg_sh.at[id0], add=True)
            return carry

        lax.fori_loop(0, steps, body, 0)
        plsc.subcore_barrier()
        pltpu.sync_copy(agg_sh.at[pl.ds(s * zr, zr)], out.at[c, pl.ds(s * zr, zr)])

    return agg


def _enc_body(HH, x_ref, w_ref, b_ref, out_ref):
    h = jnp.maximum(_dot(x_ref[...], w_ref[...]) + b_ref[...], 0.0)
    out_ref[0, :, :] = h[:, :HH]
    out_ref[1, :, :] = h[:, HH:]


def _mlp1_body(N, NBLK, HH, hf_ref, agg_ref, eps_ref, w_ref, b_ref,
               z_ref, sums_ref, s1_ref, s2_ref):
    i = pl.program_id(0)

    @pl.when(i == 0)
    def _():
        s1_ref[...] = jnp.zeros_like(s1_ref)
        s2_ref[...] = jnp.zeros_like(s2_ref)

    e = 1.0 + eps_ref[0, 0]
    z = (_dot(e * hf_ref[0] + agg_ref[0], w_ref[:HH, :])
         + _dot(e * hf_ref[1] + agg_ref[1], w_ref[HH:, :]) + b_ref[...])
    z_ref[...] = z
    mask = (lax.broadcasted_iota(jnp.int32, (BLK, 1), 0) + i * BLK) < N
    zm = jnp.where(mask, z, 0.0)
    s1_ref[...] += jnp.sum(zm, axis=0, keepdims=True)
    s2_ref[...] += jnp.sum(zm * z, axis=0, keepdims=True)

    @pl.when(i == NBLK - 1)
    def _():
        sums_ref[0:1, :] = s1_ref[...]
        sums_ref[1:2, :] = s2_ref[...]


def _bn_relu(z, sums_ref, g_ref, b_ref, N):
    m = sums_ref[0:1, :] / N
    v = sums_ref[1:2, :] / N - m * m
    inv = lax.rsqrt(v + 1e-5)
    return jnp.maximum((z - m) * inv * g_ref[...] + b_ref[...], 0.0)


def _mlp2_body(N, NBLK, z1_ref, sums1_ref, g_ref, be_ref, w_ref, b_ref,
               z_ref, sums_ref, s1_ref, s2_ref):
    i = pl.program_id(0)

    @pl.when(i == 0)
    def _():
        s1_ref[...] = jnp.zeros_like(s1_ref)
        s2_ref[...] = jnp.zeros_like(s2_ref)

    a = _bn_relu(z1_ref[...], sums1_ref, g_ref, be_ref, N)
    z = _dot(a, w_ref[...]) + b_ref[...]
    z_ref[...] = z
    mask = (lax.broadcasted_iota(jnp.int32, (BLK, 1), 0) + i * BLK) < N
    zm = jnp.where(mask, z, 0.0)
    s1_ref[...] += jnp.sum(zm, axis=0, keepdims=True)
    s2_ref[...] += jnp.sum(zm * z, axis=0, keepdims=True)

    @pl.when(i == NBLK - 1)
    def _():
        sums_ref[0:1, :] = s1_ref[...]
        sums_ref[1:2, :] = s2_ref[...]


def _hnext_body(N, HH, z2_ref, sums_ref, g_ref, b_ref, out_ref):
    h = _bn_relu(z2_ref[...], sums_ref, g_ref, b_ref, N)
    out_ref[0, :, :] = h[:, :HH]
    out_ref[1, :, :] = h[:, HH:]


def _pool_body(N, NBLK, z2_ref, sums_ref, g_ref, b_ref, batch_ref,
               wo_ref, bo_ref, out_ref, acc_ref):
    i = pl.program_id(0)

    @pl.when(i == 0)
    def _():
        acc_ref[...] = jnp.zeros_like(acc_ref)

    h = _bn_relu(z2_ref[...], sums_ref, g_ref, b_ref, N)
    gi = lax.broadcasted_iota(jnp.int32, (G_POOL, 1), 0)
    msk = (gi == batch_ref[0]).astype(F32)
    acc_ref[...] += _dot(msk, h)

    @pl.when(i == NBLK - 1)
    def _():
        out_ref[...] = _dot(acc_ref[...], wo_ref[...]) + bo_ref[...]


def _row_spec(w):
    return pl.BlockSpec((BLK, w), lambda i: (i, 0))


def _full_spec(*shape):
    nd = len(shape)
    return pl.BlockSpec(shape, lambda i: (0,) * nd)


def _split_spec(HH):
    return pl.BlockSpec((2, BLK, HH), lambda i: (0, i, 0))


def kernel(x, edge_index, batch, W_enc, b_enc, eps, W1, b1, g1, be1,
           W2, b2, g_bn, b_bn, W_out, b_out):
    N, D = x.shape
    H = W_enc.shape[1]
    HH = H // 2
    H2 = W1.shape[2]
    L = W1.shape[0]
    OUT = W_out.shape[1]
    E = edge_index.shape[1]

    NP = ((N + BLK - 1) // BLK) * BLK
    NBLK = NP // BLK
    EP = ((E + 4096 - 1) // 4096) * 4096
    steps = EP // 2048
    if steps % 32 == 0:
        EP += 4096
        steps = EP // 2048

    src = edge_index[0]
    dst = edge_index[1]
    src_p = jnp.concatenate([src, jnp.zeros((EP - E,), jnp.int32)])
    pad_dst = N + jnp.arange(EP - E, dtype=jnp.int32) % (NP - N)
    dst_p = jnp.concatenate([dst, pad_dst])
    srcs = jnp.stack([src_p, src_p + NP]).reshape(2, EP // 128, 128)
    dsts = dst_p.reshape(EP // 128, 128)
    zeros = jnp.zeros((NP // 16, HH), F32)
    x_pad = jnp.pad(x, ((0, NP - N), (0, 0)))
    batch2d = jnp.pad(batch, (0, NP - N), constant_values=G_POOL).reshape(NBLK, 1, BLK)

    grid = (NBLK,)
    cp = pltpu.CompilerParams(dimension_semantics=("arbitrary",))

    hf = pl.pallas_call(
        functools.partial(_enc_body, HH),
        grid=grid,
        in_specs=[_row_spec(D), _full_spec(D, H), _full_spec(1, H)],
        out_specs=_split_spec(HH),
        out_shape=jax.ShapeDtypeStruct((2, NP, HH), F32),
        compiler_params=cp,
    )(x_pad, W_enc, b_enc.reshape(1, H))

    agg_fn = _make_agg(NP, steps, HH)

    for l in range(L):
        agg = agg_fn(hf.reshape(2 * NP, HH), srcs, dsts, zeros)

        z1, sums1 = pl.pallas_call(
            functools.partial(_mlp1_body, N, NBLK, HH),
            grid=grid,
            in_specs=[_split_spec(HH), _split_spec(HH), _full_spec(1, 1),
                      _full_spec(H, H2), _full_spec(1, H2)],
            out_specs=[_row_spec(H2), _full_spec(2, H2)],
            out_shape=[jax.ShapeDtypeStruct((NP, H2), F32),
                       jax.ShapeDtypeStruct((2, H2), F32)],
            scratch_shapes=[pltpu.VMEM((1, H2), F32), pltpu.VMEM((1, H2), F32)],
            compiler_params=cp,
        )(hf, agg, eps[l].reshape(1, 1), W1[l], b1[l].reshape(1, H2))

        z2, sums2 = pl.pallas_call(
            functools.partial(_mlp2_body, N, NBLK),
            grid=grid,
            in_specs=[_row_spec(H2), _full_spec(2, H2), _full_spec(1, H2),
                      _full_spec(1, H2), _full_spec(H2, H), _full_spec(1, H)],
            out_specs=[_row_spec(H), _full_spec(2, H)],
            out_shape=[jax.ShapeDtypeStruct((NP, H), F32),
                       jax.ShapeDtypeStruct((2, H), F32)],
            scratch_shapes=[pltpu.VMEM((1, H), F32), pltpu.VMEM((1, H), F32)],
            compiler_params=cp,
        )(z1, sums1, g1[l].reshape(1, H2), be1[l].reshape(1, H2),
          W2[l], b2[l].reshape(1, H))

        if l < L - 1:
            hf = pl.pallas_call(
                functools.partial(_hnext_body, N, HH),
                grid=grid,
                in_specs=[_row_spec(H), _full_spec(2, H), _full_spec(1, H),
                          _full_spec(1, H)],
                out_specs=_split_spec(HH),
                out_shape=jax.ShapeDtypeStruct((2, NP, HH), F32),
                compiler_params=cp,
            )(z2, sums2, g_bn[l].reshape(1, H), b_bn[l].reshape(1, H))
        else:
            out = pl.pallas_call(
                functools.partial(_pool_body, N, NBLK),
                grid=grid,
                in_specs=[_row_spec(H), _full_spec(2, H), _full_spec(1, H),
                          _full_spec(1, H), pl.BlockSpec((1, 1, BLK), lambda i: (i, 0, 0)),
                          _full_spec(H, OUT), _full_spec(1, OUT)],
                out_specs=_full_spec(G_POOL, OUT),
                out_shape=jax.ShapeDtypeStruct((G_POOL, OUT), F32),
                scratch_shapes=[pltpu.VMEM((G_POOL, H), F32)],
                compiler_params=cp,
            )(z2, sums2, g_bn[l].reshape(1, H), b_bn[l].reshape(1, H),
              batch2d, W_out, b_out.reshape(1, OUT))

    return out


# speedup vs baseline: 1.3026x; 1.3026x over previous
import functools

import jax
import jax.numpy as jnp
from jax import lax
from jax.experimental import pallas as pl
from jax.experimental.pallas import tpu as pltpu
from jax.experimental.pallas import tpu_sc as plsc

BLK = 256
G_POOL = 64
F32 = jnp.float32


def _dot(a, b):
    return jnp.dot(a, b, preferred_element_type=F32)


@functools.lru_cache(maxsize=None)
def _make_agg(NP, steps, HH):
    mesh = plsc.VectorSubcoreMesh(core_axis_name="c", subcore_axis_name="s")
    zr = NP // 16

    @functools.partial(
        pl.kernel,
        out_type=jax.ShapeDtypeStruct((2, NP, HH), F32),
        mesh=mesh,
        scratch_types=[
            pltpu.VMEM((128,), jnp.int32),
            pltpu.VMEM((128,), jnp.int32),
            pltpu.VMEM((128, HH), F32),
            pltpu.VMEM_SHARED((NP, HH), F32),
            pltpu.SemaphoreType.DMA,
        ],
    )
    def agg(hf, srcs, dsts, zeros, out, is0, id0, rows0, agg_sh, gs0):
        c = lax.axis_index("c")
        s = lax.axis_index("s")
        pltpu.sync_copy(zeros, agg_sh.at[pl.ds(s * zr, zr)])

        plsc.subcore_barrier()

        def body(g, carry):
            row = s * steps + g
            pltpu.sync_copy(srcs.at[c, row], is0)
            pltpu.sync_copy(dsts.at[row], id0)
            pltpu.async_copy(hf.at[is0], rows0, gs0).wait()
            pltpu.sync_copy(rows0, agg_sh.at[id0], add=True)
            return carry

        lax.fori_loop(0, steps, body, 0)
        plsc.subcore_barrier()
        pltpu.sync_copy(agg_sh.at[pl.ds(s * zr, zr)], out.at[c, pl.ds(s * zr, zr)])

    return agg


def _enc_body(HH, x_ref, w_ref, b_ref, out_ref):
    h = jnp.maximum(_dot(x_ref[...], w_ref[...]) + b_ref[...], 0.0)
    out_ref[0, :, :] = h[:, :HH]
    out_ref[1, :, :] = h[:, HH:]


def _mlp1_body(N, NBLK, HH, hf_ref, agg_ref, eps_ref, w_ref, b_ref,
               z_ref, sums_ref, s1_ref, s2_ref):
    i = pl.program_id(0)

    @pl.when(i == 0)
    def _():
        s1_ref[...] = jnp.zeros_like(s1_ref)
        s2_ref[...] = jnp.zeros_like(s2_ref)

    e = 1.0 + eps_ref[0, 0]
    z = (_dot(e * hf_ref[0] + agg_ref[0], w_ref[:HH, :])
         + _dot(e * hf_ref[1] + agg_ref[1], w_ref[HH:, :]) + b_ref[...])
    z_ref[...] = z
    mask = (lax.broadcasted_iota(jnp.int32, (BLK, 1), 0) + i * BLK) < N
    zm = jnp.where(mask, z, 0.0)
    s1_ref[...] += jnp.sum(zm, axis=0, keepdims=True)
    s2_ref[...] += jnp.sum(zm * z, axis=0, keepdims=True)

    @pl.when(i == NBLK - 1)
    def _():
        sums_ref[0:1, :] = s1_ref[...]
        sums_ref[1:2, :] = s2_ref[...]


def _bn_relu(z, sums_ref, g_ref, b_ref, N):
    m = sums_ref[0:1, :] / N
    v = sums_ref[1:2, :] / N - m * m
    inv = lax.rsqrt(v + 1e-5)
    return jnp.maximum((z - m) * inv * g_ref[...] + b_ref[...], 0.0)


def _mlp2_body(N, NBLK, z1_ref, sums1_ref, g_ref, be_ref, w_ref, b_ref,
               z_ref, sums_ref, s1_ref, s2_ref):
    i = pl.program_id(0)

    @pl.when(i == 0)
    def _():
        s1_ref[...] = jnp.zeros_like(s1_ref)
        s2_ref[...] = jnp.zeros_like(s2_ref)

    a = _bn_relu(z1_ref[...], sums1_ref, g_ref, be_ref, N)
    z = _dot(a, w_ref[...]) + b_ref[...]
    z_ref[...] = z
    mask = (lax.broadcasted_iota(jnp.int32, (BLK, 1), 0) + i * BLK) < N
    zm = jnp.where(mask, z, 0.0)
    s1_ref[...] += jnp.sum(zm, axis=0, keepdims=True)
    s2_ref[...] += jnp.sum(zm * z, axis=0, keepdims=True)

    @pl.when(i == NBLK - 1)
    def _():
        sums_ref[0:1, :] = s1_ref[...]
        sums_ref[1:2, :] = s2_ref[...]


def _hnext_body(N, HH, z2_ref, sums_ref, g_ref, b_ref, out_ref):
    h = _bn_relu(z2_ref[...], sums_ref, g_ref, b_ref, N)
    out_ref[0, :, :] = h[:, :HH]
    out_ref[1, :, :] = h[:, HH:]


def _pool_body(N, NBLK, z2_ref, sums_ref, g_ref, b_ref, batch_ref,
               wo_ref, bo_ref, out_ref, acc_ref):
    i = pl.program_id(0)

    @pl.when(i == 0)
    def _():
        acc_ref[...] = jnp.zeros_like(acc_ref)

    h = _bn_relu(z2_ref[...], sums_ref, g_ref, b_ref, N)
    gi = lax.broadcasted_iota(jnp.int32, (G_POOL, 1), 0)
    msk = (gi == batch_ref[0]).astype(F32)
    acc_ref[...] += _dot(msk, h)

    @pl.when(i == NBLK - 1)
    def _():
        out_ref[...] = _dot(acc_ref[...], wo_ref[...]) + bo_ref[...]


def _row_spec(w):
    return pl.BlockSpec((BLK, w), lambda i: (i, 0))


def _full_spec(*shape):
    nd = len(shape)
    return pl.BlockSpec(shape, lambda i: (0,) * nd)


def _split_spec(HH):
    return pl.BlockSpec((2, BLK, HH), lambda i: (0, i, 0))


def kernel(x, edge_index, batch, W_enc, b_enc, eps, W1, b1, g1, be1,
           W2, b2, g_bn, b_bn, W_out, b_out):
    N, D = x.shape
    H = W_enc.shape[1]
    HH = H // 2
    H2 = W1.shape[2]
    L = W1.shape[0]
    OUT = W_out.shape[1]
    E = edge_index.shape[1]

    NP = ((N + BLK - 1) // BLK) * BLK
    NBLK = NP // BLK
    EP = ((E + 4096 - 1) // 4096) * 4096
    steps = EP // 2048
    if steps % 32 == 0:
        EP += 4096
        steps = EP // 2048

    src = edge_index[0]
    dst = edge_index[1]
    E16 = ((E + 15) // 16) * 16
    if E16 != E:
        src = jnp.concatenate([src, src[: E16 - E]])
        dst = jnp.concatenate([dst, N + jnp.arange(E16 - E, dtype=jnp.int32)])
    per = EP // 16
    real = E16 // 16
    src16 = src.reshape(16, real)
    dst16 = dst.reshape(16, real)
    pad_src = src16[:, : per - real]
    pad_dst = N + (jnp.arange(16 * (per - real), dtype=jnp.int32)
                   % (NP - N)).reshape(16, per - real)
    src_p = jnp.concatenate([src16, pad_src], axis=1).reshape(EP)
    dst_p = jnp.concatenate([dst16, pad_dst], axis=1).reshape(EP)
    srcs = jnp.stack([src_p, src_p + NP]).reshape(2, EP // 128, 128)
    dsts = dst_p.reshape(EP // 128, 128)
    zeros = jnp.zeros((NP // 16, HH), F32)
    x_pad = jnp.pad(x, ((0, NP - N), (0, 0)))
    batch2d = jnp.pad(batch, (0, NP - N), constant_values=G_POOL).reshape(NBLK, 1, BLK)

    grid = (NBLK,)
    cp = pltpu.CompilerParams(dimension_semantics=("arbitrary",))

    hf = pl.pallas_call(
        functools.partial(_enc_body, HH),
        grid=grid,
        in_specs=[_row_spec(D), _full_spec(D, H), _full_spec(1, H)],
        out_specs=_split_spec(HH),
        out_shape=jax.ShapeDtypeStruct((2, NP, HH), F32),
        compiler_params=cp,
    )(x_pad, W_enc, b_enc.reshape(1, H))

    agg_fn = _make_agg(NP, steps, HH)

    for l in range(L):
        agg = agg_fn(hf.reshape(2 * NP, HH), srcs, dsts, zeros)

        z1, sums1 = pl.pallas_call(
            functools.partial(_mlp1_body, N, NBLK, HH),
            grid=grid,
            in_specs=[_split_spec(HH), _split_spec(HH), _full_spec(1, 1),
                      _full_spec(H, H2), _full_spec(1, H2)],
            out_specs=[_row_spec(H2), _full_spec(2, H2)],
            out_shape=[jax.ShapeDtypeStruct((NP, H2), F32),
                       jax.ShapeDtypeStruct((2, H2), F32)],
            scratch_shapes=[pltpu.VMEM((1, H2), F32), pltpu.VMEM((1, H2), F32)],
            compiler_params=cp,
        )(hf, agg, eps[l].reshape(1, 1), W1[l], b1[l].reshape(1, H2))

        z2, sums2 = pl.pallas_call(
            functools.partial(_mlp2_body, N, NBLK),
            grid=grid,
            in_specs=[_row_spec(H2), _full_spec(2, H2), _full_spec(1, H2),
                      _full_spec(1, H2), _full_spec(H2, H), _full_spec(1, H)],
            out_specs=[_row_spec(H), _full_spec(2, H)],
            out_shape=[jax.ShapeDtypeStruct((NP, H), F32),
                       jax.ShapeDtypeStruct((2, H), F32)],
            scratch_shapes=[pltpu.VMEM((1, H), F32), pltpu.VMEM((1, H), F32)],
            compiler_params=cp,
        )(z1, sums1, g1[l].reshape(1, H2), be1[l].reshape(1, H2),
          W2[l], b2[l].reshape(1, H))

        if l < L - 1:
            hf = pl.pallas_call(
                functools.partial(_hnext_body, N, HH),
                grid=grid,
                in_specs=[_row_spec(H), _full_spec(2, H), _full_spec(1, H),
                          _full_spec(1, H)],
                out_specs=_split_spec(HH),
                out_shape=jax.ShapeDtypeStruct((2, NP, HH), F32),
                compiler_params=cp,
            )(z2, sums2, g_bn[l].reshape(1, H), b_bn[l].reshape(1, H))
        else:
            out = pl.pallas_call(
                functools.partial(_pool_body, N, NBLK),
                grid=grid,
                in_specs=[_row_spec(H), _full_spec(2, H), _full_spec(1, H),
                          _full_spec(1, H), pl.BlockSpec((1, 1, BLK), lambda i: (i, 0, 0)),
                          _full_spec(H, OUT), _full_spec(1, OUT)],
                out_specs=_full_spec(G_POOL, OUT),
                out_shape=jax.ShapeDtypeStruct((G_POOL, OUT), F32),
                scratch_shapes=[pltpu.VMEM((G_POOL, H), F32)],
                compiler_params=cp,
            )(z2, sums2, g_bn[l].reshape(1, H), b_bn[l].reshape(1, H),
              batch2d, W_out, b_out.reshape(1, OUT))

    return out


# speedup vs baseline: 1.9423x; 1.4911x over previous
import functools

import jax
import jax.numpy as jnp
from jax import lax
from jax.experimental import pallas as pl
from jax.experimental.pallas import tpu as pltpu
from jax.experimental.pallas import tpu_sc as plsc

BLK = 256
G_POOL = 64
F32 = jnp.float32


def _dot(a, b):
    return jnp.dot(a, b, preferred_element_type=F32)


@functools.lru_cache(maxsize=None)
def _make_agg(NP, steps, HH):
    mesh = plsc.VectorSubcoreMesh(core_axis_name="c", subcore_axis_name="s")
    zr = NP // 16

    @functools.partial(
        pl.kernel,
        out_type=jax.ShapeDtypeStruct((2, NP, HH), F32),
        mesh=mesh,
        scratch_types=[
            pltpu.VMEM((128,), jnp.int32),
            pltpu.VMEM((128,), jnp.int32),
            pltpu.VMEM((128,), jnp.int32),
            pltpu.VMEM((128,), jnp.int32),
            pltpu.VMEM((128, HH), F32),
            pltpu.VMEM((128, HH), F32),
            pltpu.VMEM_SHARED((NP, HH), F32),
            pltpu.SemaphoreType.DMA,
            pltpu.SemaphoreType.DMA,
        ],
    )
    def agg(hf, srcs, dsts, zeros, out,
            is0, is1, id0, id1, rows0, rows1, agg_sh, gs0, gs1):
        c = lax.axis_index("c")
        s = lax.axis_index("s")
        pltpu.sync_copy(zeros, agg_sh.at[pl.ds(s * zr, zr)])

        def ildx(g, isb, idb):
            row = s * steps + g
            pltpu.sync_copy(srcs.at[c, row], isb)
            pltpu.sync_copy(dsts.at[row], idb)

        ildx(0, is0, id0)
        plsc.subcore_barrier()
        pltpu.async_copy(hf.at[is0], rows0, gs0)

        def body(p, carry):
            g = 2 * p
            ildx(g + 1, is1, id1)
            pltpu.async_copy(hf.at[is1], rows1, gs1)
            pltpu.make_async_copy(hf.at[is0], rows0, gs0).wait()
            pltpu.sync_copy(rows0, agg_sh.at[id0], add=True)

            @pl.when(g + 2 < steps)
            def _():
                ildx(g + 2, is0, id0)
                pltpu.async_copy(hf.at[is0], rows0, gs0)

            pltpu.make_async_copy(hf.at[is1], rows1, gs1).wait()
            pltpu.sync_copy(rows1, agg_sh.at[id1], add=True)
            return carry

        lax.fori_loop(0, steps // 2, body, 0)
        plsc.subcore_barrier()
        pltpu.sync_copy(agg_sh.at[pl.ds(s * zr, zr)], out.at[c, pl.ds(s * zr, zr)])

    return agg


def _enc_body(HH, x_ref, w_ref, b_ref, out_ref):
    h = jnp.maximum(_dot(x_ref[...], w_ref[...]) + b_ref[...], 0.0)
    out_ref[0, :, :] = h[:, :HH]
    out_ref[1, :, :] = h[:, HH:]


def _mlp1_body(N, NBLK, HH, hf_ref, agg_ref, eps_ref, w_ref, b_ref,
               z_ref, sums_ref, s1_ref, s2_ref):
    i = pl.program_id(0)

    @pl.when(i == 0)
    def _():
        s1_ref[...] = jnp.zeros_like(s1_ref)
        s2_ref[...] = jnp.zeros_like(s2_ref)

    e = 1.0 + eps_ref[0, 0]
    z = (_dot(e * hf_ref[0] + agg_ref[0], w_ref[:HH, :])
         + _dot(e * hf_ref[1] + agg_ref[1], w_ref[HH:, :]) + b_ref[...])
    z_ref[...] = z
    mask = (lax.broadcasted_iota(jnp.int32, (BLK, 1), 0) + i * BLK) < N
    zm = jnp.where(mask, z, 0.0)
    s1_ref[...] += jnp.sum(zm, axis=0, keepdims=True)
    s2_ref[...] += jnp.sum(zm * z, axis=0, keepdims=True)

    @pl.when(i == NBLK - 1)
    def _():
        sums_ref[0:1, :] = s1_ref[...]
        sums_ref[1:2, :] = s2_ref[...]


def _bn_relu(z, sums_ref, g_ref, b_ref, N):
    m = sums_ref[0:1, :] / N
    v = sums_ref[1:2, :] / N - m * m
    inv = lax.rsqrt(v + 1e-5)
    return jnp.maximum((z - m) * inv * g_ref[...] + b_ref[...], 0.0)


def _mlp2_body(N, NBLK, z1_ref, sums1_ref, g_ref, be_ref, w_ref, b_ref,
               z_ref, sums_ref, s1_ref, s2_ref):
    i = pl.program_id(0)

    @pl.when(i == 0)
    def _():
        s1_ref[...] = jnp.zeros_like(s1_ref)
        s2_ref[...] = jnp.zeros_like(s2_ref)

    a = _bn_relu(z1_ref[...], sums1_ref, g_ref, be_ref, N)
    z = _dot(a, w_ref[...]) + b_ref[...]
    z_ref[...] = z
    mask = (lax.broadcasted_iota(jnp.int32, (BLK, 1), 0) + i * BLK) < N
    zm = jnp.where(mask, z, 0.0)
    s1_ref[...] += jnp.sum(zm, axis=0, keepdims=True)
    s2_ref[...] += jnp.sum(zm * z, axis=0, keepdims=True)

    @pl.when(i == NBLK - 1)
    def _():
        sums_ref[0:1, :] = s1_ref[...]
        sums_ref[1:2, :] = s2_ref[...]


def _hnext_body(N, HH, z2_ref, sums_ref, g_ref, b_ref, out_ref):
    h = _bn_relu(z2_ref[...], sums_ref, g_ref, b_ref, N)
    out_ref[0, :, :] = h[:, :HH]
    out_ref[1, :, :] = h[:, HH:]


def _pool_body(N, NBLK, z2_ref, sums_ref, g_ref, b_ref, batch_ref,
               wo_ref, bo_ref, out_ref, acc_ref):
    i = pl.program_id(0)

    @pl.when(i == 0)
    def _():
        acc_ref[...] = jnp.zeros_like(acc_ref)

    h = _bn_relu(z2_ref[...], sums_ref, g_ref, b_ref, N)
    gi = lax.broadcasted_iota(jnp.int32, (G_POOL, 1), 0)
    msk = (gi == batch_ref[0]).astype(F32)
    acc_ref[...] += _dot(msk, h)

    @pl.when(i == NBLK - 1)
    def _():
        out_ref[...] = _dot(acc_ref[...], wo_ref[...]) + bo_ref[...]


def _row_spec(w):
    return pl.BlockSpec((BLK, w), lambda i: (i, 0))


def _full_spec(*shape):
    nd = len(shape)
    return pl.BlockSpec(shape, lambda i: (0,) * nd)


def _split_spec(HH):
    return pl.BlockSpec((2, BLK, HH), lambda i: (0, i, 0))


def kernel(x, edge_index, batch, W_enc, b_enc, eps, W1, b1, g1, be1,
           W2, b2, g_bn, b_bn, W_out, b_out):
    N, D = x.shape
    H = W_enc.shape[1]
    HH = H // 2
    H2 = W1.shape[2]
    L = W1.shape[0]
    OUT = W_out.shape[1]
    E = edge_index.shape[1]

    NP = ((N + BLK - 1) // BLK) * BLK
    NBLK = NP // BLK
    EP = ((E + 4096 - 1) // 4096) * 4096
    steps = EP // 2048
    if steps % 32 == 0:
        EP += 4096
        steps = EP // 2048

    src = edge_index[0]
    dst = edge_index[1]
    E16 = ((E + 15) // 16) * 16
    if E16 != E:
        src = jnp.concatenate([src, src[: E16 - E]])
        dst = jnp.concatenate([dst, N + jnp.arange(E16 - E, dtype=jnp.int32)])
    per = EP // 16
    real = E16 // 16
    src16 = src.reshape(16, real)
    dst16 = dst.reshape(16, real)
    pad_src = src16[:, : per - real]
    pad_dst = N + (jnp.arange(16 * (per - real), dtype=jnp.int32)
                   % (NP - N)).reshape(16, per - real)
    src_p = jnp.concatenate([src16, pad_src], axis=1).reshape(EP)
    dst_p = jnp.concatenate([dst16, pad_dst], axis=1).reshape(EP)
    srcs = jnp.stack([src_p, src_p + NP]).reshape(2, EP // 128, 128)
    dsts = dst_p.reshape(EP // 128, 128)
    zeros = jnp.zeros((NP // 16, HH), F32)
    x_pad = jnp.pad(x, ((0, NP - N), (0, 0)))
    batch2d = jnp.pad(batch, (0, NP - N), constant_values=G_POOL).reshape(NBLK, 1, BLK)

    grid = (NBLK,)
    cp = pltpu.CompilerParams(dimension_semantics=("arbitrary",))

    hf = pl.pallas_call(
        functools.partial(_enc_body, HH),
        grid=grid,
        in_specs=[_row_spec(D), _full_spec(D, H), _full_spec(1, H)],
        out_specs=_split_spec(HH),
        out_shape=jax.ShapeDtypeStruct((2, NP, HH), F32),
        compiler_params=cp,
    )(x_pad, W_enc, b_enc.reshape(1, H))

    agg_fn = _make_agg(NP, steps, HH)

    for l in range(L):
        agg = agg_fn(hf.reshape(2 * NP, HH), srcs, dsts, zeros)

        z1, sums1 = pl.pallas_call(
            functools.partial(_mlp1_body, N, NBLK, HH),
            grid=grid,
            in_specs=[_split_spec(HH), _split_spec(HH), _full_spec(1, 1),
                      _full_spec(H, H2), _full_spec(1, H2)],
            out_specs=[_row_spec(H2), _full_spec(2, H2)],
            out_shape=[jax.ShapeDtypeStruct((NP, H2), F32),
                       jax.ShapeDtypeStruct((2, H2), F32)],
            scratch_shapes=[pltpu.VMEM((1, H2), F32), pltpu.VMEM((1, H2), F32)],
            compiler_params=cp,
        )(hf, agg, eps[l].reshape(1, 1), W1[l], b1[l].reshape(1, H2))

        z2, sums2 = pl.pallas_call(
            functools.partial(_mlp2_body, N, NBLK),
            grid=grid,
            in_specs=[_row_spec(H2), _full_spec(2, H2), _full_spec(1, H2),
                      _full_spec(1, H2), _full_spec(H2, H), _full_spec(1, H)],
            out_specs=[_row_spec(H), _full_spec(2, H)],
            out_shape=[jax.ShapeDtypeStruct((NP, H), F32),
                       jax.ShapeDtypeStruct((2, H), F32)],
            scratch_shapes=[pltpu.VMEM((1, H), F32), pltpu.VMEM((1, H), F32)],
            compiler_params=cp,
        )(z1, sums1, g1[l].reshape(1, H2), be1[l].reshape(1, H2),
          W2[l], b2[l].reshape(1, H))

        if l < L - 1:
            hf = pl.pallas_call(
                functools.partial(_hnext_body, N, HH),
                grid=grid,
                in_specs=[_row_spec(H), _full_spec(2, H), _full_spec(1, H),
                          _full_spec(1, H)],
                out_specs=_split_spec(HH),
                out_shape=jax.ShapeDtypeStruct((2, NP, HH), F32),
                compiler_params=cp,
            )(z2, sums2, g_bn[l].reshape(1, H), b_bn[l].reshape(1, H))
        else:
            out = pl.pallas_call(
                functools.partial(_pool_body, N, NBLK),
                grid=grid,
                in_specs=[_row_spec(H), _full_spec(2, H), _full_spec(1, H),
                          _full_spec(1, H), pl.BlockSpec((1, 1, BLK), lambda i: (i, 0, 0)),
                          _full_spec(H, OUT), _full_spec(1, OUT)],
                out_specs=_full_spec(G_POOL, OUT),
                out_shape=jax.ShapeDtypeStruct((G_POOL, OUT), F32),
                scratch_shapes=[pltpu.VMEM((G_POOL, H), F32)],
                compiler_params=cp,
            )(z2, sums2, g_bn[l].reshape(1, H), b_bn[l].reshape(1, H),
              batch2d, W_out, b_out.reshape(1, OUT))

    return out
